# baseline (speedup 1.0000x reference)
import jax
import jax.numpy as jnp
from jax import lax
from jax.experimental import pallas as pl
from jax.experimental.pallas import tpu as pltpu
from jax.experimental.pallas import tpu_sc as plsc

_N = 10000
_D = 256
_DH = 128
_DQ = 64
_NT = 16
_CH = 128
_CHUNKS = 80
_EPT = _CH * _CHUNKS
_EPAD = _EPT * _NT
_RPT = _N // _NT
_IR = 125

_BM = 1000


def _mm_body(x_ref, w_ref, o_ref):
    o_ref[0] = jnp.dot(x_ref[...], w_ref[...], preferred_element_type=jnp.float32)


def _support_split(x, weight):
    return pl.pallas_call(
        _mm_body,
        grid=(_N // _BM, 2),
        in_specs=[
            pl.BlockSpec((_BM, _D), lambda i, j: (i, 0)),
            pl.BlockSpec((_D, _DH), lambda i, j: (0, j)),
        ],
        out_specs=pl.BlockSpec((1, _BM, _DH), lambda i, j: (j, i, 0)),
        out_shape=jax.ShapeDtypeStruct((2, _N, _DH), jnp.float32),
    )(x, weight)


def _sc_body(sup_ref, col_ref, row_ref, adj_ref, bias_ref, out_ref,
             col_v, row_v, adj_v, cidx_v, rows_v, bias_v, init_v, acc_sh,
             g0, g1, g2, g3, s0, s1, s2, s3):
    c = lax.axis_index("c")
    s = lax.axis_index("s")

    pltpu.sync_copy(col_ref.at[s], col_v)
    pltpu.sync_copy(row_ref.at[s], row_v)
    pltpu.sync_copy(adj_ref.at[s], adj_v)
    pltpu.sync_copy(bias_ref, bias_v)

    gsems = (g0, g1, g2, g3)
    ssems = (s0, s1, s2, s3)

    for p in range(2):
        q = 2 * c + p

        base = (c * _N) * 2 + p

        def cidx_chunk(j, carry):
            for g in range(_CH // 16):
                sl = pl.ds(g * 16, 16)
                cidx_v[j, sl] = col_v[j, sl] * 2 + base
            return carry

        lax.fori_loop(0, _CHUNKS, cidx_chunk, 0)

        def init_row(r, carry):
            for g in range(_DQ // 16):
                init_v[r, pl.ds(g * 16, 16)] = bias_v[q, pl.ds(g * 16, 16)]
            return carry

        lax.fori_loop(0, _IR, init_row, 0)
        for t in range(_RPT // _IR):
            pltpu.sync_copy(init_v, acc_sh.at[pl.ds(s * _RPT + t * _IR, _IR)])
        plsc.subcore_barrier()

        def g_start(j, b):
            return pltpu.async_copy(
                sup_ref.at[cidx_v.at[j]], rows_v.at[b], gsems[b])

        def g_wait(j, b):
            pltpu.make_async_copy(
                sup_ref.at[cidx_v.at[j]], rows_v.at[b], gsems[b]).wait()

        def s_start(j, b):
            return pltpu.async_copy(
                rows_v.at[b], acc_sh.at[row_v.at[j]], ssems[b], add=True)

        def s_wait(j, b):
            pltpu.make_async_copy(
                rows_v.at[b], acc_sh.at[row_v.at[j]], ssems[b]).wait()

        g_start(0, 0)
        g_start(1, 1)

        def chunk_body(jj, carry):
            for b in range(4):
                j = 4 * jj + b
                g_wait(j, b)

                def scale16(g, inner):
                    a16 = adj_v[j, pl.ds(g * 16, 16)]
                    for l in range(16):
                        a = a16[l]
                        e = g * 16 + l
                        for gg in range(_DQ // 16):
                            sl = pl.ds(gg * 16, 16)
                            rows_v[b, e, sl] = rows_v[b, e, sl] * a
                    return inner

                lax.fori_loop(0, _CH // 16, scale16, 0)

                s_start(j, b)

                bn = (b + 2) % 4

                @pl.when(j >= 2)
                def _():
                    s_wait(j - 2, bn)

                @pl.when(j + 2 < _CHUNKS)
                def _():
                    g_start(j + 2, bn)
            return carry

        lax.fori_loop(0, _CHUNKS // 4, chunk_body, 0)
        s_wait(_CHUNKS - 2, (_CHUNKS - 2) % 4)
        s_wait(_CHUNKS - 1, (_CHUNKS - 1) % 4)

        plsc.subcore_barrier()
        pltpu.sync_copy(acc_sh.at[pl.ds(s * _RPT, _RPT)],
                        out_ref.at[pl.ds(s * _RPT, _RPT), q])


_sc_agg = pl.kernel(
    _sc_body,
    out_type=jax.ShapeDtypeStruct((_N, 4, _DQ), jnp.float32),
    mesh=plsc.VectorSubcoreMesh(
        core_axis_name="c", subcore_axis_name="s", num_cores=2, num_subcores=_NT),
    compiler_params=pltpu.CompilerParams(use_tc_tiling_on_sc=False),
    scratch_types=[
        pltpu.VMEM((_CHUNKS, _CH), jnp.int32),
        pltpu.VMEM((_CHUNKS, _CH), jnp.int32),
        pltpu.VMEM((_CHUNKS, _CH), jnp.float32),
        pltpu.VMEM((_CHUNKS, _CH), jnp.int32),
        pltpu.VMEM((4, _CH, _DQ), jnp.float32),
        pltpu.VMEM((4, _DQ), jnp.float32),
        pltpu.VMEM((_IR, _DQ), jnp.float32),
        pltpu.VMEM_SHARED((_N, _DQ), jnp.float32),
        pltpu.SemaphoreType.DMA,
        pltpu.SemaphoreType.DMA,
        pltpu.SemaphoreType.DMA,
        pltpu.SemaphoreType.DMA,
        pltpu.SemaphoreType.DMA,
        pltpu.SemaphoreType.DMA,
        pltpu.SemaphoreType.DMA,
        pltpu.SemaphoreType.DMA,
    ],
)


def kernel(x, edge_index, adj_vals, weight, bias):
    support2 = _support_split(x, weight)
    sup_rows = support2.reshape(4 * _N, _DQ)
    row = edge_index[0].astype(jnp.int32)
    col = edge_index[1].astype(jnp.int32)
    adj = adj_vals.astype(jnp.float32)
    pad = _EPAD - row.shape[0]
    zi = jnp.zeros((pad,), jnp.int32)
    col3 = jnp.concatenate([col, zi]).reshape(_NT, _CHUNKS, _CH)
    row3 = jnp.concatenate([row, zi]).reshape(_NT, _CHUNKS, _CH)
    adj3 = jnp.concatenate([adj, jnp.zeros((pad,), jnp.float32)]).reshape(
        _NT, _CHUNKS, _CH)
    bias4 = bias.reshape(4, _DQ)
    out4 = _sc_agg(sup_rows, col3, row3, adj3, bias4)
    return out4.reshape(_N, _D)


# speedup vs baseline: 1.0673x; 1.0673x over previous
import jax
import jax.numpy as jnp
from jax import lax
from jax.experimental import pallas as pl
from jax.experimental.pallas import tpu as pltpu
from jax.experimental.pallas import tpu_sc as plsc

_N = 10000
_D = 256
_DH = 128
_DQ = 64
_NT = 16
_CH = 64
_CHUNKS = 160
_EPT = _CH * _CHUNKS
_EPAD = _EPT * _NT
_RPT = _N // _NT
_IR = 125

_BM = 1000


def _mm_body(x_ref, w_ref, o_ref):
    o_ref[0] = jnp.dot(x_ref[...], w_ref[...], preferred_element_type=jnp.float32)


def _support_split(x, weight):
    return pl.pallas_call(
        _mm_body,
        grid=(_N // _BM, 2),
        in_specs=[
            pl.BlockSpec((_BM, _D), lambda i, j: (i, 0)),
            pl.BlockSpec((_D, _DH), lambda i, j: (0, j)),
        ],
        out_specs=pl.BlockSpec((1, _BM, _DH), lambda i, j: (j, i, 0)),
        out_shape=jax.ShapeDtypeStruct((2, _N, _DH), jnp.float32),
    )(x, weight)


def _sc_body(sup_ref, col_ref, row_ref, adj_ref, bias_ref, out_ref,
             col_v, row_v, adj_v, cidx_v, rows_v, bias_v, init_v, acc_sh,
             g0, g1, g2, g3, g4, g5, g6, g7, s0, s1, s2, s3):
    c = lax.axis_index("c")
    s = lax.axis_index("s")

    pltpu.sync_copy(col_ref.at[s], col_v)
    pltpu.sync_copy(row_ref.at[s], row_v)
    pltpu.sync_copy(adj_ref.at[s], adj_v)
    pltpu.sync_copy(bias_ref, bias_v)

    gsems = (g0, g1, g2, g3, g4, g5, g6, g7)
    ssems = (s0, s1, s2, s3)

    for p in range(2):
        q = 2 * c + p

        base = (c * _N) * 2 + p

        def cidx_chunk(j, carry):
            for g in range(_CH // 16):
                sl = pl.ds(g * 16, 16)
                cidx_v[j, sl] = col_v[j, sl] * 2 + base
            return carry

        lax.fori_loop(0, _CHUNKS, cidx_chunk, 0)

        def init_row(r, carry):
            for g in range(_DQ // 16):
                init_v[r, pl.ds(g * 16, 16)] = bias_v[q, pl.ds(g * 16, 16)]
            return carry

        lax.fori_loop(0, _IR, init_row, 0)
        for t in range(_RPT // _IR):
            pltpu.sync_copy(init_v, acc_sh.at[pl.ds(s * _RPT + t * _IR, _IR)])
        plsc.subcore_barrier()

        def g_start(j, b):
            return pltpu.async_copy(
                sup_ref.at[cidx_v.at[j]], rows_v.at[b], gsems[b])

        def g_wait(j, b):
            pltpu.make_async_copy(
                sup_ref.at[cidx_v.at[j]], rows_v.at[b], gsems[b]).wait()

        def s_start(j, b):
            return pltpu.async_copy(
                rows_v.at[b % 8], acc_sh.at[row_v.at[j]], ssems[b % 4],
                add=True)

        def s_wait(j, b):
            pltpu.make_async_copy(
                rows_v.at[b % 8], acc_sh.at[row_v.at[j]], ssems[b % 4]).wait()

        for b in range(4):
            g_start(b, b)

        def chunk_body(jj, carry):
            for b in range(8):
                j = 8 * jj + b
                g_wait(j, b)

                def scale16(g, inner):
                    a16 = adj_v[j, pl.ds(g * 16, 16)]
                    for l in range(16):
                        a = a16[l]
                        e = g * 16 + l
                        for gg in range(_DQ // 16):
                            sl = pl.ds(gg * 16, 16)
                            rows_v[b, e, sl] = rows_v[b, e, sl] * a
                    return inner

                lax.fori_loop(0, _CH // 16, scale16, 0)

                @pl.when(j >= 4)
                def _():
                    s_wait(j - 4, b + 4)

                s_start(j, b)

                @pl.when(j + 4 < _CHUNKS)
                def _():
                    g_start(j + 4, (b + 4) % 8)
            return carry

        lax.fori_loop(0, _CHUNKS // 8, chunk_body, 0)
        for j in range(_CHUNKS - 4, _CHUNKS):
            s_wait(j, j % 4)

        plsc.subcore_barrier()
        pltpu.sync_copy(acc_sh.at[pl.ds(s * _RPT, _RPT)],
                        out_ref.at[pl.ds(s * _RPT, _RPT), q])


_sc_agg = pl.kernel(
    _sc_body,
    out_type=jax.ShapeDtypeStruct((_N, 4, _DQ), jnp.float32),
    mesh=plsc.VectorSubcoreMesh(
        core_axis_name="c", subcore_axis_name="s", num_cores=2, num_subcores=_NT),
    compiler_params=pltpu.CompilerParams(use_tc_tiling_on_sc=False),
    scratch_types=[
        pltpu.VMEM((_CHUNKS, _CH), jnp.int32),
        pltpu.VMEM((_CHUNKS, _CH), jnp.int32),
        pltpu.VMEM((_CHUNKS, _CH), jnp.float32),
        pltpu.VMEM((_CHUNKS, _CH), jnp.int32),
        pltpu.VMEM((8, _CH, _DQ), jnp.float32),
        pltpu.VMEM((4, _DQ), jnp.float32),
        pltpu.VMEM((_IR, _DQ), jnp.float32),
        pltpu.VMEM_SHARED((_N, _DQ), jnp.float32),
        pltpu.SemaphoreType.DMA,
        pltpu.SemaphoreType.DMA,
        pltpu.SemaphoreType.DMA,
        pltpu.SemaphoreType.DMA,
        pltpu.SemaphoreType.DMA,
        pltpu.SemaphoreType.DMA,
        pltpu.SemaphoreType.DMA,
        pltpu.SemaphoreType.DMA,
        pltpu.SemaphoreType.DMA,
        pltpu.SemaphoreType.DMA,
        pltpu.SemaphoreType.DMA,
        pltpu.SemaphoreType.DMA,
    ],
)


def kernel(x, edge_index, adj_vals, weight, bias):
    support2 = _support_split(x, weight)
    sup_rows = support2.reshape(4 * _N, _DQ)
    row = edge_index[0].astype(jnp.int32)
    col = edge_index[1].astype(jnp.int32)
    adj = adj_vals.astype(jnp.float32)
    pad = _EPAD - row.shape[0]
    zi = jnp.zeros((pad,), jnp.int32)
    col3 = jnp.concatenate([col, zi]).reshape(_NT, _CHUNKS, _CH)
    row3 = jnp.concatenate([row, zi]).reshape(_NT, _CHUNKS, _CH)
    adj3 = jnp.concatenate([adj, jnp.zeros((pad,), jnp.float32)]).reshape(
        _NT, _CHUNKS, _CH)
    bias4 = bias.reshape(4, _DQ)
    out4 = _sc_agg(sup_rows, col3, row3, adj3, bias4)
    return out4.reshape(_N, _D)
